# baseline (speedup 1.0000x reference)
import functools

import jax
import jax.numpy as jnp
from jax import lax
from jax.experimental import pallas as pl
from jax.experimental.pallas import tpu as pltpu
from jax.experimental.pallas import tpu_sc as plsc

NC = 2
NS = 16
NW = NC * NS

NBUF = 4
K = 2


@functools.partial(jax.jit, static_argnames=("batch", "hist", "emb_dim"))
def _gather_rows(tok_t, weight, *, batch, hist, emb_dim):
    bpw = batch // NW

    mesh = plsc.VectorSubcoreMesh(core_axis_name="c", subcore_axis_name="s")

    @functools.partial(
        pl.kernel,
        mesh=mesh,
        compiler_params=pltpu.CompilerParams(use_tc_tiling_on_sc=False),
        out_type=jax.ShapeDtypeStruct((batch, hist, emb_dim), jnp.float32),
        scratch_types=[
            pltpu.VMEM((hist, bpw), jnp.int32),
            pltpu.VMEM((NBUF, bpw, emb_dim), jnp.float32),
            pltpu.SemaphoreType.DMA,
            pltpu.SemaphoreType.DMA,
        ],
    )
    def k(tok_hbm, table_hbm, out_hbm, idx_hv, rows_v, gsem, ssem):
        wid = lax.axis_index("s") * NC + lax.axis_index("c")
        b0 = wid * bpw
        pltpu.sync_copy(tok_hbm.at[:, pl.ds(b0, bpw)], idx_hv)

        def gather(h):
            return pltpu.make_async_copy(
                table_hbm.at[idx_hv.at[h]], rows_v.at[h % NBUF], gsem)

        def store(h):
            return pltpu.make_async_copy(
                rows_v.at[h % NBUF], out_hbm.at[pl.ds(b0, bpw), h], ssem)

        for t in range(K):
            gather(t).start()
        for h in range(hist):
            gather(h).wait()
            store(h).start()
            f = h + K
            if f < hist:
                if f >= NBUF:
                    store(f - NBUF).wait()
                gather(f).start()
        for h in range(max(hist - NBUF, 0), hist):
            store(h).wait()

    return k(tok_t, weight)


def kernel(token, weight):
    batch, hist = token.shape
    vocab, emb_dim = weight.shape
    return _gather_rows(token.T, weight, batch=batch, hist=hist,
                        emb_dim=emb_dim)


# speedup vs baseline: 1.0538x; 1.0538x over previous
import functools

import jax
import jax.numpy as jnp
from jax import lax
from jax.experimental import pallas as pl
from jax.experimental.pallas import tpu as pltpu
from jax.experimental.pallas import tpu_sc as plsc

NC = 2
NS = 16
NW = NC * NS

NBUF = 4
K = 2


@functools.partial(jax.jit, static_argnames=("batch", "hist", "emb_dim"))
def _gather_rows(tok_t, weight, *, batch, hist, emb_dim):
    bpw = batch // NW

    mesh = plsc.VectorSubcoreMesh(core_axis_name="c", subcore_axis_name="s")

    @functools.partial(
        pl.kernel,
        mesh=mesh,
        compiler_params=pltpu.CompilerParams(use_tc_tiling_on_sc=False),
        out_type=jax.ShapeDtypeStruct((batch, hist * emb_dim), jnp.float32),
        scratch_types=[
            pltpu.VMEM((hist, bpw), jnp.int32),
            pltpu.VMEM((NBUF, bpw, emb_dim), jnp.float32),
            pltpu.SemaphoreType.DMA,
            pltpu.SemaphoreType.DMA,
        ],
    )
    def k(tok_hbm, table_hbm, out_hbm, idx_hv, rows_v, gsem, ssem):
        wid = lax.axis_index("s") * NC + lax.axis_index("c")
        b0 = wid * bpw
        pltpu.sync_copy(tok_hbm.at[:, pl.ds(b0, bpw)], idx_hv)

        def gather(h):
            return pltpu.make_async_copy(
                table_hbm.at[idx_hv.at[h]], rows_v.at[h % NBUF], gsem)

        def store(h):
            return pltpu.make_async_copy(
                rows_v.at[h % NBUF],
                out_hbm.at[pl.ds(b0, bpw), pl.ds(h * emb_dim, emb_dim)],
                ssem)

        for t in range(K):
            gather(t).start()
        for h in range(hist):
            gather(h).wait()
            store(h).start()
            f = h + K
            if f < hist:
                if f >= NBUF:
                    store(f - NBUF).wait()
                gather(f).start()
        for h in range(max(hist - NBUF, 0), hist):
            store(h).wait()

    return k(tok_t, weight)


def kernel(token, weight):
    batch, hist = token.shape
    vocab, emb_dim = weight.shape
    out2 = _gather_rows(token.T, weight, batch=batch, hist=hist,
                        emb_dim=emb_dim)
    return out2.reshape(batch, hist, emb_dim)


# speedup vs baseline: 1.0540x; 1.0002x over previous
import functools

import jax
import jax.numpy as jnp
from jax import lax
from jax.experimental import pallas as pl
from jax.experimental.pallas import tpu as pltpu
from jax.experimental.pallas import tpu_sc as plsc

NC = 2
NS = 16
NW = NC * NS

NBUF = 4
K = 2


@functools.partial(jax.jit, static_argnames=("batch", "hist", "emb_dim"))
def _gather_rows(tok_t, weight, *, batch, hist, emb_dim):
    bpw = batch // NW

    mesh = plsc.VectorSubcoreMesh(core_axis_name="c", subcore_axis_name="s")

    @functools.partial(
        pl.kernel,
        mesh=mesh,
        compiler_params=pltpu.CompilerParams(use_tc_tiling_on_sc=False,
                                             needs_layout_passes=False),
        out_type=jax.ShapeDtypeStruct((batch, hist * emb_dim), jnp.float32),
        scratch_types=[
            pltpu.VMEM((hist, bpw), jnp.int32),
            pltpu.VMEM((NBUF, bpw, emb_dim), jnp.float32),
            pltpu.SemaphoreType.DMA,
            pltpu.SemaphoreType.DMA,
        ],
    )
    def k(tok_hbm, table_hbm, out_hbm, idx_hv, rows_v, gsem, ssem):
        wid = lax.axis_index("s") * NC + lax.axis_index("c")
        b0 = wid * bpw
        pltpu.sync_copy(tok_hbm.at[:, pl.ds(b0, bpw)], idx_hv)

        def gather(h):
            return pltpu.make_async_copy(
                table_hbm.at[idx_hv.at[h]], rows_v.at[h % NBUF], gsem)

        def store(h):
            return pltpu.make_async_copy(
                rows_v.at[h % NBUF],
                out_hbm.at[pl.ds(b0, bpw), pl.ds(h * emb_dim, emb_dim)],
                ssem)

        for t in range(K):
            gather(t).start()
        for h in range(hist):
            gather(h).wait()
            store(h).start()
            f = h + K
            if f < hist:
                if f >= NBUF:
                    store(f - NBUF).wait()
                gather(f).start()
        for h in range(max(hist - NBUF, 0), hist):
            store(h).wait()

    return k(tok_t, weight)


def kernel(token, weight):
    batch, hist = token.shape
    vocab, emb_dim = weight.shape
    out2 = _gather_rows(token.T, weight, batch=batch, hist=hist,
                        emb_dim=emb_dim)
    return out2.reshape(batch, hist, emb_dim)
